# baseline (speedup 1.0000x reference)
import functools

import jax
import jax.numpy as jnp
import numpy as np
from jax import lax
from jax.experimental import pallas as pl
from jax.experimental.pallas import tpu as pltpu
from jax.experimental.pallas import tpu_sc as plsc

B, L, C = 4, 8192, 768
K = 4916
R, LN = 64, 128
EPS = 1e-6

NC, NS = 2, 16
NW = NC * NS

W2, NWIN2 = 8, 20
_STARTS = np.array([154 * w if w < 20 else 3080 + 153 * (w - 20)
                    for w in range(NW)])
_NPOS = np.array([154 if w < 20 else 153 for w in range(NW)])
_SW = np.minimum(W2 * np.arange(NWIN2)[None, :], (_NPOS - W2)[:, None])
_POS = (_STARTS[:, None, None] + _SW[:, :, None]
        + np.arange(W2)[None, None, :])
_SIDX = (_POS[:, :, None, :] * 6
         + np.arange(6)[None, None, :, None]).astype(np.int32)
_POS_FLAT = _POS.reshape(-1)

_P0R, _P0L = (K - 1) // LN, (K - 1) % LN
_P1R, _P1L = K // LN, K % LN


def _xor_partner(x, iflat, d):
    if d < LN:
        t, ax = d, 2
    else:
        t, ax = d // LN, 1
    fwd = jnp.roll(x, -t, axis=ax)
    bwd = jnp.roll(x, t, axis=ax)
    return jnp.where((iflat & d) == 0, fwd, bwd)


def _sort_body(score_ref, gidx_ref, mask_ref, w_ref):
    score = score_ref[...]

    iflat = (lax.broadcasted_iota(jnp.int32, (B, R, LN), 1) * LN
             + lax.broadcasted_iota(jnp.int32, (B, R, LN), 2))

    ns = -score
    ids = iflat
    s = 2
    while s <= L:
        asc = (iflat & s) == 0
        d = s // 2
        while d >= 1:
            pns = _xor_partner(ns, iflat, d)
            pid = _xor_partner(ids, iflat, d)
            low = (iflat & d) == 0
            take_min = asc == low
            less = (ns < pns) | ((ns == pns) & (ids < pid))
            keep_self = take_min == less
            ns = jnp.where(keep_self, ns, pns)
            ids = jnp.where(keep_self, ids, pid)
            d //= 2
        s *= 2

    thr = -ns[:, _P0R:_P0R + 1, _P0L:_P0L + 1]
    istar = ids[:, _P0R:_P0R + 1, _P0L:_P0L + 1]
    m = -ns[:, _P1R:_P1R + 1, _P1L:_P1L + 1]

    keep = (score > thr) | ((score == thr) & (iflat <= istar))
    mask = keep.astype(jnp.float32)
    e = jnp.exp(score - m) * (1.0 - mask)
    z = jnp.sum(jnp.sum(e, axis=2, keepdims=True), axis=1, keepdims=True)

    gidx_ref[...] = ids + lax.broadcasted_iota(jnp.int32, (B, R, LN), 0) * L
    mask_ref[...] = mask
    w_ref[...] = e / z


_sort_call = pl.pallas_call(
    _sort_body,
    out_shape=[
        jax.ShapeDtypeStruct((B, R, LN), jnp.int32),
        jax.ShapeDtypeStruct((B, R, LN), jnp.float32),
        jax.ShapeDtypeStruct((B, R, LN), jnp.float32),
    ],
)


@functools.cache
def _make_sc_gather():
    mesh = plsc.VectorSubcoreMesh(
        core_axis_name="c", subcore_axis_name="s", num_cores=NC)

    @functools.partial(
        pl.kernel,
        mesh=mesh,
        out_type=jax.ShapeDtypeStruct((K * 6, 4, 128), jnp.float32),
        scratch_types=[
            pltpu.VMEM((NWIN2, 4, W2), jnp.int32),
            pltpu.VMEM((NWIN2, 6 * W2), jnp.int32),
            pltpu.VMEM((4, W2, C), jnp.float32),
            pltpu.VMEM((4, W2, C), jnp.float32),
            pltpu.VMEM((6 * W2, 4, 128), jnp.float32),
            pltpu.VMEM((6 * W2, 4, 128), jnp.float32),
            pltpu.SemaphoreType.DMA,
            pltpu.SemaphoreType.DMA,
            pltpu.SemaphoreType.DMA,
            pltpu.SemaphoreType.DMA,
        ],
    )
    def _sc_gather(table_hbm, gix_hbm, six_hbm, out_hbm,
                   gixv, sixv, bufA0, bufA1, bufB0, bufB1,
                   gs0, gs1, ws0, ws1):
        wid = lax.axis_index("s") * NC + lax.axis_index("c")
        pltpu.sync_copy(gix_hbm.at[wid], gixv)
        pltpu.sync_copy(six_hbm.at[wid], sixv)
        bufA = (bufA0, bufA1)
        bufB = (bufB0, bufB1)
        gsem = (gs0, gs1)
        wsem = (ws0, ws1)

        def issue_gathers(w, p):
            for b in range(4):
                pltpu.async_copy(table_hbm.at[gixv.at[w, b]],
                                 bufA[p].at[b], gsem[p])

        def drain_gathers(w, p):
            for b in range(4):
                pltpu.make_async_copy(table_hbm.at[gixv.at[w, b]],
                                      bufA[p].at[b], gsem[p]).wait()

        def issue_scatters(w, p):
            pltpu.async_copy(bufB[p], out_hbm.at[sixv.at[w]], wsem[p])

        def drain_scatters(w, p):
            pltpu.make_async_copy(bufB[p], out_hbm.at[sixv.at[w]],
                                  wsem[p]).wait()

        def rearrange(bA, bB):
            def rbody(j, carry):
                for b in range(4):
                    for ct in range(6):
                        for k in range(8):
                            bB[ct * W2 + j, b, pl.ds(k * 16, 16)] = (
                                bA[b, j, pl.ds(ct * 128 + k * 16, 16)])
                return carry
            lax.fori_loop(0, W2, rbody, 0, unroll=2)

        def window(w, p):
            drain_gathers(w, p)

            @pl.when(w >= 2)
            def _():
                drain_scatters(w - 2, p)
            rearrange(bufA[p], bufB[p])

            @pl.when(w + 2 < NWIN2)
            def _():
                issue_gathers(w + 2, p)
            issue_scatters(w, p)

        issue_gathers(0, 0)
        issue_gathers(1, 1)

        def body(i, carry):
            window(2 * i, 0)
            window(2 * i + 1, 1)
            return carry
        lax.fori_loop(0, NWIN2 // 2, body, 0)
        drain_scatters(NWIN2 - 2, 0)
        drain_scatters(NWIN2 - 1, 1)

    return _sc_gather


CH = 2048
NCH = L // CH


def _extra_body(tok_ref, w_ref, out_ref):
    @pl.when(pl.program_id(1) == 0)
    def _():
        out_ref[...] = jnp.zeros_like(out_ref)
    wrow = w_ref[0, 0]
    tok = tok_ref[0]
    acc = lax.dot_general(wrow, tok, (((1,), (0,)), ((), ())),
                          preferred_element_type=jnp.float32)
    out_ref[...] += acc.reshape(1, 1, C)


_extra_call = pl.pallas_call(
    _extra_body,
    grid=(B, NCH),
    in_specs=[
        pl.BlockSpec((1, CH, C), lambda b, c: (b, c, 0)),
        pl.BlockSpec((1, 1, 1, CH), lambda b, c: (b, c, 0, 0)),
    ],
    out_specs=pl.BlockSpec((1, 1, C), lambda b, c: (b, 0, 0)),
    out_shape=jax.ShapeDtypeStruct((B, 1, C), jnp.float32),
)


def kernel(tokens, attention_x, attention_y):
    coverage = attention_y.mean(axis=1) / (attention_x.mean(axis=1) + EPS)
    alpha = jax.nn.sigmoid(1.0 - coverage)
    beta = 1.0 - alpha
    score = alpha[:, None] * attention_x + beta[:, None] * attention_y
    gidx3, mask3, w3 = _sort_call(score.reshape(B, R, LN))

    kidx = gidx3.reshape(B, L)[:, :K]
    gix = (jnp.take(kidx, _POS_FLAT, axis=1)
           .reshape(B, NW, NWIN2, W2).transpose(1, 2, 0, 3))
    out_rows = _make_sc_gather()(tokens.reshape(B * L, C), gix,
                                 jnp.asarray(_SIDX.reshape(NW, NWIN2, 6 * W2)))
    select_tokens = lax.reshape(out_rows.reshape(K, 6, 4, 128),
                                (B, K, C), dimensions=(2, 0, 1, 3))

    extra_token = _extra_call(tokens, w3.reshape(B, NCH, 1, CH))
    return select_tokens, extra_token, mask3.reshape(B, L)


# speedup vs baseline: 1.2899x; 1.2899x over previous
import functools

import jax
import jax.numpy as jnp
import numpy as np
from jax import lax
from jax.experimental import pallas as pl
from jax.experimental.pallas import tpu as pltpu
from jax.experimental.pallas import tpu_sc as plsc

B, L, C = 4, 8192, 768
K = 4916
R, LN = 64, 128
EPS = 1e-6

NC, NS = 2, 16
NW = NC * NS

W2, NWIN2 = 8, 20
_STARTS = np.array([154 * w if w < 20 else 3080 + 153 * (w - 20)
                    for w in range(NW)])
_NPOS = np.array([154 if w < 20 else 153 for w in range(NW)])
_SW = np.minimum(W2 * np.arange(NWIN2)[None, :], (_NPOS - W2)[:, None])
_POS = (_STARTS[:, None, None] + _SW[:, :, None]
        + np.arange(W2)[None, None, :])
_SIDX = ((_POS[:, :, None, None, :] * 6
          + np.arange(6)[None, None, None, :, None]) * 4
         + np.arange(4)[None, None, :, None, None]).astype(np.int32)
_POS_FLAT = _POS.reshape(-1)

_P0R, _P0L = (K - 1) // LN, (K - 1) % LN
_P1R, _P1L = K // LN, K % LN


def _xor_partner(x, iflat, d):
    if d < LN:
        t, ax = d, 2
    else:
        t, ax = d // LN, 1
    fwd = jnp.roll(x, -t, axis=ax)
    bwd = jnp.roll(x, t, axis=ax)
    return jnp.where((iflat & d) == 0, fwd, bwd)


def _sort_body(score_ref, gidx_ref, mask_ref, w_ref):
    score = score_ref[...]

    iflat = (lax.broadcasted_iota(jnp.int32, (B, R, LN), 1) * LN
             + lax.broadcasted_iota(jnp.int32, (B, R, LN), 2))

    ns = -score
    ids = iflat
    s = 2
    while s <= L:
        asc = (iflat & s) == 0
        d = s // 2
        while d >= 1:
            pns = _xor_partner(ns, iflat, d)
            pid = _xor_partner(ids, iflat, d)
            low = (iflat & d) == 0
            take_min = asc == low
            less = (ns < pns) | ((ns == pns) & (ids < pid))
            keep_self = take_min == less
            ns = jnp.where(keep_self, ns, pns)
            ids = jnp.where(keep_self, ids, pid)
            d //= 2
        s *= 2

    thr = -ns[:, _P0R:_P0R + 1, _P0L:_P0L + 1]
    istar = ids[:, _P0R:_P0R + 1, _P0L:_P0L + 1]
    m = -ns[:, _P1R:_P1R + 1, _P1L:_P1L + 1]

    keep = (score > thr) | ((score == thr) & (iflat <= istar))
    mask = keep.astype(jnp.float32)
    e = jnp.exp(score - m) * (1.0 - mask)
    z = jnp.sum(jnp.sum(e, axis=2, keepdims=True), axis=1, keepdims=True)

    gidx_ref[...] = ids + lax.broadcasted_iota(jnp.int32, (B, R, LN), 0) * L
    mask_ref[...] = mask
    w_ref[...] = e / z


_sort_call = pl.pallas_call(
    _sort_body,
    out_shape=[
        jax.ShapeDtypeStruct((B, R, LN), jnp.int32),
        jax.ShapeDtypeStruct((B, R, LN), jnp.float32),
        jax.ShapeDtypeStruct((B, R, LN), jnp.float32),
    ],
)


@functools.cache
def _make_sc_gather():
    mesh = plsc.VectorSubcoreMesh(
        core_axis_name="c", subcore_axis_name="s", num_cores=NC)

    @functools.partial(
        pl.kernel,
        mesh=mesh,
        out_type=jax.ShapeDtypeStruct((K * 24, 128), jnp.float32),
        scratch_types=[
            pltpu.VMEM((NWIN2, 4, W2), jnp.int32),
            pltpu.VMEM((NWIN2, 4, 6 * W2), jnp.int32),
            pltpu.VMEM((4, W2, C), jnp.float32),
            pltpu.VMEM((4, W2, C), jnp.float32),
            pltpu.VMEM((4, 6 * W2, 128), jnp.float32),
            pltpu.VMEM((4, 6 * W2, 128), jnp.float32),
            pltpu.SemaphoreType.DMA,
            pltpu.SemaphoreType.DMA,
            pltpu.SemaphoreType.DMA,
            pltpu.SemaphoreType.DMA,
        ],
    )
    def _sc_gather(table_hbm, gix_hbm, six_hbm, out_hbm,
                   gixv, sixv, bufA0, bufA1, bufB0, bufB1,
                   gs0, gs1, ws0, ws1):
        wid = lax.axis_index("s") * NC + lax.axis_index("c")
        pltpu.sync_copy(gix_hbm.at[wid], gixv)
        pltpu.sync_copy(six_hbm.at[wid], sixv)
        bufA = (bufA0, bufA1)
        bufB = (bufB0, bufB1)
        gsem = (gs0, gs1)
        wsem = (ws0, ws1)

        def issue_gathers(w, p):
            for b in range(4):
                pltpu.async_copy(table_hbm.at[gixv.at[w, b]],
                                 bufA[p].at[b], gsem[p])

        def drain_gathers(w, p):
            for b in range(4):
                pltpu.make_async_copy(table_hbm.at[gixv.at[w, b]],
                                      bufA[p].at[b], gsem[p]).wait()

        def issue_scatters(w, p):
            for b in range(4):
                pltpu.async_copy(bufB[p].at[b], out_hbm.at[sixv.at[w, b]],
                                 wsem[p])

        def drain_scatters(w, p):
            for b in range(4):
                pltpu.make_async_copy(bufB[p].at[b],
                                      out_hbm.at[sixv.at[w, b]],
                                      wsem[p]).wait()

        def rearrange(bA, bB):
            def rbody(j, carry):
                for b in range(4):
                    for ct in range(6):
                        for k in range(8):
                            bB[b, ct * W2 + j, pl.ds(k * 16, 16)] = (
                                bA[b, j, pl.ds(ct * 128 + k * 16, 16)])
                return carry
            lax.fori_loop(0, W2, rbody, 0, unroll=2)

        def window(w, p):
            drain_gathers(w, p)

            @pl.when(w >= 2)
            def _():
                drain_scatters(w - 2, p)
            rearrange(bufA[p], bufB[p])

            @pl.when(w + 2 < NWIN2)
            def _():
                issue_gathers(w + 2, p)
            issue_scatters(w, p)

        issue_gathers(0, 0)
        issue_gathers(1, 1)

        def body(i, carry):
            window(2 * i, 0)
            window(2 * i + 1, 1)
            return carry
        lax.fori_loop(0, NWIN2 // 2, body, 0)
        drain_scatters(NWIN2 - 2, 0)
        drain_scatters(NWIN2 - 1, 1)

    return _sc_gather


CH = 2048
NCH = L // CH


def _extra_body(tok_ref, w_ref, out_ref):
    @pl.when(pl.program_id(1) == 0)
    def _():
        out_ref[...] = jnp.zeros_like(out_ref)
    wrow = w_ref[0, 0]
    tok = tok_ref[0]
    acc = lax.dot_general(wrow, tok, (((1,), (0,)), ((), ())),
                          preferred_element_type=jnp.float32)
    out_ref[...] += acc.reshape(1, 1, C)


_extra_call = pl.pallas_call(
    _extra_body,
    grid=(B, NCH),
    in_specs=[
        pl.BlockSpec((1, CH, C), lambda b, c: (b, c, 0)),
        pl.BlockSpec((1, 1, 1, CH), lambda b, c: (b, c, 0, 0)),
    ],
    out_specs=pl.BlockSpec((1, 1, C), lambda b, c: (b, 0, 0)),
    out_shape=jax.ShapeDtypeStruct((B, 1, C), jnp.float32),
)


def kernel(tokens, attention_x, attention_y):
    coverage = attention_y.mean(axis=1) / (attention_x.mean(axis=1) + EPS)
    alpha = jax.nn.sigmoid(1.0 - coverage)
    beta = 1.0 - alpha
    score = alpha[:, None] * attention_x + beta[:, None] * attention_y
    gidx3, mask3, w3 = _sort_call(score.reshape(B, R, LN))

    kidx = gidx3.reshape(B, L)[:, :K]
    gix = (jnp.take(kidx, _POS_FLAT, axis=1)
           .reshape(B, NW, NWIN2, W2).transpose(1, 2, 0, 3))
    out_rows = _make_sc_gather()(tokens.reshape(B * L, C), gix,
                                 jnp.asarray(_SIDX.reshape(NW, NWIN2, 4, 6 * W2)))
    select_tokens = lax.reshape(out_rows.reshape(K, 6, 4, 128),
                                (B, K, C), dimensions=(2, 0, 1, 3))

    extra_token = _extra_call(tokens, w3.reshape(B, NCH, 1, CH))
    return select_tokens, extra_token, mask3.reshape(B, L)
